# baseline (speedup 1.0000x reference)
import functools

import jax
import jax.numpy as jnp
from jax import lax
from jax.experimental import pallas as pl
from jax.experimental.pallas import tpu as pltpu
from jax.experimental.pallas import tpu_sc as plsc

N_AUTHOR, N_PAPER, N_UNIT = 50000, 100000, 5000
D_IN, H = 128, 32
NEG = -1e30

NC, NS, LANES = 2, 16, 16
NW = NC * NS
CB = 128

BIG_ACC = 50176
SMALL_ACC = 6144


def _src_proj(x, W, A):
    N = x.shape[0]
    R = 1000

    def body(x_ref, w_ref, a_ref, hs_ref, as_ref):
        hs = jnp.dot(x_ref[...], w_ref[...],
                     preferred_element_type=jnp.float32)
        hs_ref[...] = hs
        as_ref[...] = jnp.sum(hs * a_ref[...], axis=1, keepdims=True)

    return pl.pallas_call(
        body,
        grid=(N // R,),
        in_specs=[
            pl.BlockSpec((R, D_IN), lambda i: (i, 0)),
            pl.BlockSpec((D_IN, H), lambda i: (0, 0)),
            pl.BlockSpec((1, H), lambda i: (0, 0)),
        ],
        out_specs=[
            pl.BlockSpec((R, H), lambda i: (i, 0)),
            pl.BlockSpec((R, 1), lambda i: (i, 0)),
        ],
        out_shape=[
            jax.ShapeDtypeStruct((N, H), jnp.float32),
            jax.ShapeDtypeStruct((N, 1), jnp.float32),
        ],
    )(x, W, A)


def _dst_proj(x, wv):
    N = x.shape[0]
    R = 1000

    def body(x_ref, w_ref, ad_ref):
        ad_ref[...] = jnp.dot(x_ref[...], w_ref[...],
                              preferred_element_type=jnp.float32)

    return pl.pallas_call(
        body,
        grid=(N // R,),
        in_specs=[
            pl.BlockSpec((R, D_IN), lambda i: (i, 0)),
            pl.BlockSpec((D_IN, 1), lambda i: (0, 0)),
        ],
        out_specs=pl.BlockSpec((R, 1), lambda i: (i, 0)),
        out_shape=jax.ShapeDtypeStruct((N, 1), jnp.float32),
    )(x, wv)


NBUF = 2


@functools.lru_cache(maxsize=None)
def _mega_kernel(steps4):
    mesh = plsc.VectorSubcoreMesh(core_axis_name="c", subcore_axis_name="s")
    big_chunks = BIG_ACC // CB
    sml_chunks = SMALL_ACC // CB
    big_iters = -(-big_chunks // NS)
    sml_iters = -(-sml_chunks // NS)

    def body(*args):
        srcs, dsts = args[0:4], args[4:8]
        ass, ads, hss = args[8:12], args[12:16], args[16:20]
        m_h = args[20]
        nums, dens = args[21:25], args[25:29]
        scr = args[29:]
        si = scr[0:NBUF]
        di = scr[NBUF:2 * NBUF]
        asv = scr[2 * NBUF:3 * NBUF]
        adv = scr[3 * NBUF:4 * NBUF]
        exv = scr[4 * NBUF:5 * NBUF]
        rows = scr[5 * NBUF:6 * NBUF]
        (zrow, zden, mv, num_big, den_big,
         num_sml, den_sml) = scr[6 * NBUF:6 * NBUF + 7]
        sems = scr[6 * NBUF + 7:]
        ga = sems[0:NBUF]
        gd = sems[NBUF:2 * NBUF]
        gr = sems[2 * NBUF:3 * NBUF]
        sd = sems[3 * NBUF:4 * NBUF]
        sn = sems[4 * NBUF:5 * NBUF]
        cid = lax.axis_index("c")
        sid = lax.axis_index("s")
        wid = sid * NC + cid

        z16 = jnp.zeros((LANES,), jnp.float32)

        def zb(i, _):
            zrow[i, pl.ds(0, LANES)] = z16
            zrow[i, pl.ds(LANES, LANES)] = z16
            return _

        lax.fori_loop(0, CB, zb, None)
        for j in range(CB // LANES):
            zden[pl.ds(j * LANES, LANES)] = z16
        pltpu.sync_copy(m_h, mv)

        def _chunk_loop(iters, total, fn):
            def it(k, _):
                ch = sid + k * NS

                @pl.when(ch < total)
                def _do():
                    fn(ch * CB)

                return _

            lax.fori_loop(0, iters, it, None)

        def zero_accs():
            def z1(r):
                pltpu.sync_copy(zrow, num_big.at[pl.ds(r, CB)])
                pltpu.sync_copy(zden, den_big.at[pl.ds(r, CB)])

            def z2(r):
                pltpu.sync_copy(zrow, num_sml.at[pl.ds(r, CB)])
                pltpu.sync_copy(zden, den_sml.at[pl.ds(r, CB)])

            _chunk_loop(big_iters, big_chunks, z1)
            _chunk_loop(sml_iters, sml_chunks, z2)

        def copy_out(num_sh, den_sh, num_o, den_o, iters, total):
            def co(r):
                pltpu.sync_copy(num_sh.at[pl.ds(r, CB)],
                                num_o.at[cid, pl.ds(r, CB)])
                pltpu.sync_copy(den_sh.at[pl.ds(r, CB)],
                                den_o.at[cid, pl.ds(r, CB)])

            _chunk_loop(iters, total, co)

        def edge_phase(rel, num_sh, den_sh):
            steps = steps4[rel]
            src_h, dst_h = srcs[rel], dsts[rel]
            as_h, ad_h, hs_h = ass[rel], ads[rel], hss[rel]
            mvec = mv[pl.ds(rel * LANES, LANES)]
            wbase = wid * (steps * CB)

            def gathers_start(b, base):
                pltpu.sync_copy(src_h.at[pl.ds(base, CB)], si[b])
                pltpu.sync_copy(dst_h.at[pl.ds(base, CB)], di[b])
                pltpu.async_copy(as_h.at[si[b]], asv[b], ga[b])
                pltpu.async_copy(ad_h.at[di[b]], adv[b], gd[b])
                pltpu.async_copy(hs_h.at[si[b]], rows[b], gr[b])

            def gathers_wait(b):
                pltpu.make_async_copy(as_h.at[si[b]], asv[b], ga[b]).wait()
                pltpu.make_async_copy(ad_h.at[di[b]], adv[b], gd[b]).wait()
                pltpu.make_async_copy(hs_h.at[si[b]], rows[b], gr[b]).wait()

            def scatters_start(b):
                pltpu.async_copy(exv[b], den_sh.at[di[b]], sd[b], add=True)
                pltpu.async_copy(rows[b], num_sh.at[di[b]], sn[b], add=True)

            def scatters_wait(b):
                pltpu.make_async_copy(exv[b], den_sh.at[di[b]], sd[b]).wait()
                pltpu.make_async_copy(rows[b], num_sh.at[di[b]], sn[b]).wait()

            def compute(b):
                def grp(j, _):
                    a16 = asv[b][pl.ds(j * LANES, LANES)]
                    d16 = adv[b][pl.ds(j * LANES, LANES)]
                    t = a16 + d16
                    e = jnp.maximum(t, 0.2 * t)
                    ex = jnp.exp(e - mvec)
                    exv[b][pl.ds(j * LANES, LANES)] = ex
                    for i in range(LANES):
                        r = j * LANES + i
                        s = ex[i]
                        rows[b][r, pl.ds(0, LANES)] = (
                            rows[b][r, pl.ds(0, LANES)] * s)
                        rows[b][r, pl.ds(LANES, LANES)] = (
                            rows[b][r, pl.ds(LANES, LANES)] * s)
                    return _

                lax.fori_loop(0, CB // LANES, grp, None)

            for b in range(NBUF - 1):
                gathers_start(b, wbase + b * CB)

            def group(p, _):
                for b in range(NBUF):
                    s = NBUF * p + b
                    nxt = (NBUF - 1 + b) % NBUF

                    @pl.when(s + NBUF - 1 < steps)
                    def _issue():
                        @pl.when(s >= 1)
                        def _drain():
                            scatters_wait(nxt)

                        gathers_start(nxt, wbase + (s + NBUF - 1) * CB)

                    gathers_wait(b)
                    compute(b)
                    scatters_start(b)
                return _

            lax.fori_loop(0, steps // NBUF, group, None)
            for b in range(NBUF):
                scatters_wait(b)

        zero_accs()
        plsc.subcore_barrier()
        edge_phase(0, num_big, den_big)
        edge_phase(1, num_sml, den_sml)
        plsc.subcore_barrier()
        copy_out(num_big, den_big, nums[0], dens[0], big_iters, big_chunks)
        copy_out(num_sml, den_sml, nums[1], dens[1], sml_iters, sml_chunks)
        zero_accs()
        plsc.subcore_barrier()
        edge_phase(2, num_big, den_big)
        edge_phase(3, num_sml, den_sml)
        plsc.subcore_barrier()
        copy_out(num_big, den_big, nums[2], dens[2], big_iters, big_chunks)
        copy_out(num_sml, den_sml, nums[3], dens[3], sml_iters, sml_chunks)

    accs = (BIG_ACC, SMALL_ACC, BIG_ACC, SMALL_ACC)
    return pl.kernel(
        body,
        compiler_params=pltpu.CompilerParams(use_tc_tiling_on_sc=False),
        out_type=(
            [jax.ShapeDtypeStruct((NC, a, H), jnp.float32) for a in accs]
            + [jax.ShapeDtypeStruct((NC, a), jnp.float32) for a in accs]
        ),
        mesh=mesh,
        scratch_types=(
            [pltpu.VMEM((CB,), jnp.int32)] * (2 * NBUF)
            + [pltpu.VMEM((CB,), jnp.float32)] * (3 * NBUF)
            + [pltpu.VMEM((CB, H), jnp.float32)] * NBUF
            + [pltpu.VMEM((CB, H), jnp.float32),
               pltpu.VMEM((CB,), jnp.float32),
               pltpu.VMEM((4 * LANES,), jnp.float32),
               pltpu.VMEM_SHARED((BIG_ACC, H), jnp.float32),
               pltpu.VMEM_SHARED((BIG_ACC,), jnp.float32),
               pltpu.VMEM_SHARED((SMALL_ACC, H), jnp.float32),
               pltpu.VMEM_SHARED((SMALL_ACC,), jnp.float32)]
            + [pltpu.SemaphoreType.DMA] * (5 * NBUF)
        ),
    )


def _pad_relation(ei, a_d, n_dst_real, n_acc):
    E = ei.shape[1]
    quant = NBUF * NW * CB
    e_pad = ((E + quant - 1) // quant) * quant
    pad = e_pad - E
    src = jnp.concatenate([ei[0].astype(jnp.int32),
                           jnp.zeros((pad,), jnp.int32)])
    dst = jnp.concatenate([ei[1].astype(jnp.int32),
                           jnp.full((pad,), n_dst_real, jnp.int32)])
    ad_pad = jnp.concatenate(
        [a_d, jnp.full((n_acc - a_d.shape[0],), NEG, jnp.float32)])
    return src, dst, ad_pad, e_pad // (NW * CB)


def _run_all(rels):
    srcs, dsts, ads, steps4, ms = [], [], [], [], []
    for ei, a_s, a_d, hs, n_dst_real, n_acc in rels:
        src, dst, ad_pad, steps = _pad_relation(ei, a_d, n_dst_real, n_acc)
        srcs.append(src)
        dsts.append(dst)
        ads.append(ad_pad)
        steps4.append(steps)
        t = jnp.max(a_s) + jnp.max(a_d)
        ms.append(jnp.full((LANES,), jnp.maximum(t, 0.2 * t), jnp.float32))
    m_all = jnp.concatenate(ms)
    k = _mega_kernel(tuple(steps4))
    outs = k(*srcs, *dsts, *[r[1] for r in rels], *ads,
             *[r[3] for r in rels], m_all)
    return list(zip(outs[0:4], outs[4:8]))


def _post_one(num, den_t, b, W_lin, b_lin):
    N = num.shape[1]
    R = 512

    def body(n_ref, d_ref, b_ref, wl_ref, bl_ref, o_ref):
        nm = n_ref[0] + n_ref[1]
        dn = d_ref[..., 0:1] + d_ref[..., 1:2]
        o = nm / (dn + 1e-16) + b_ref[...]
        o_ref[...] = jnp.dot(jnp.maximum(o, 0.0), wl_ref[...],
                             preferred_element_type=jnp.float32) + bl_ref[...]

    return pl.pallas_call(
        body,
        grid=(N // R,),
        in_specs=[
            pl.BlockSpec((NC, R, H), lambda i: (0, i, 0)),
            pl.BlockSpec((R, NC), lambda i: (i, 0)),
            pl.BlockSpec((1, H), lambda i: (0, 0)),
            pl.BlockSpec((H, H), lambda i: (0, 0)),
            pl.BlockSpec((1, H), lambda i: (0, 0)),
        ],
        out_specs=pl.BlockSpec((R, H), lambda i: (i, 0)),
        out_shape=jax.ShapeDtypeStruct((N, H), jnp.float32),
    )(num, den_t, b, W_lin, b_lin)


def _post_paper(num1, den1_t, b1, num2, den2_t, b2, W_lin, b_lin):
    R = 512
    last2 = SMALL_ACC // R - 1

    def body(n1, d1, bb1, n2, d2, bb2, wl, bl, o_ref):
        o1 = (n1[0] + n1[1]) / (d1[..., 0:1] + d1[..., 1:2] + 1e-16) + bb1[...]
        o2 = (n2[0] + n2[1]) / (d2[..., 0:1] + d2[..., 1:2] + 1e-16) + bb2[...]
        o = 0.5 * (o1 + o2)
        o_ref[...] = jnp.dot(jnp.maximum(o, 0.0), wl[...],
                             preferred_element_type=jnp.float32) + bl[...]

    return pl.pallas_call(
        body,
        grid=(BIG_ACC // R,),
        in_specs=[
            pl.BlockSpec((NC, R, H), lambda i: (0, i, 0)),
            pl.BlockSpec((R, NC), lambda i: (i, 0)),
            pl.BlockSpec((1, H), lambda i: (0, 0)),
            pl.BlockSpec((NC, R, H), lambda i: (0, jnp.minimum(i, last2), 0)),
            pl.BlockSpec((R, NC), lambda i: (jnp.minimum(i, last2), 0)),
            pl.BlockSpec((1, H), lambda i: (0, 0)),
            pl.BlockSpec((H, H), lambda i: (0, 0)),
            pl.BlockSpec((1, H), lambda i: (0, 0)),
        ],
        out_specs=pl.BlockSpec((R, H), lambda i: (i, 0)),
        out_shape=jax.ShapeDtypeStruct((BIG_ACC, H), jnp.float32),
    )(num1, den1_t, b1, num2, den2_t, b2, W_lin, b_lin)


def kernel(x_author, x_paper, x_unit,
           Ws_wr, Wd_wr, As_wr, Ad_wr, b_wr,
           Ws_pu, Wd_pu, As_pu, Ad_pu, b_pu,
           Ws_rw, Wd_rw, As_rw, Ad_rw, b_rw,
           Ws_rp, Wd_rp, As_rp, Ad_rp, b_rp,
           W_lin, b_lin,
           ei_wr, ei_pu, ei_rw, ei_rp):
    xp50 = x_paper[:50000]
    xp5 = x_paper[:5000]

    hs_wr, as_wr = _src_proj(x_author, Ws_wr, As_wr.reshape(1, H))
    hs_rw, as_rw = _src_proj(xp50, Ws_rw, As_rw.reshape(1, H))
    hs_pu, as_pu = _src_proj(xp5, Ws_pu, As_pu.reshape(1, H))
    hs_rp, as_rp = _src_proj(x_unit, Ws_rp, As_rp.reshape(1, H))
    ad_wr = _dst_proj(xp50, (Wd_wr @ Ad_wr).reshape(D_IN, 1))
    ad_rw = _dst_proj(x_author, (Wd_rw @ Ad_rw).reshape(D_IN, 1))
    ad_pu = _dst_proj(x_unit, (Wd_pu @ Ad_pu).reshape(D_IN, 1))
    ad_rp = _dst_proj(xp5, (Wd_rp @ Ad_rp).reshape(D_IN, 1))

    parts = _run_all([
        (ei_wr, as_wr[:, 0], ad_wr[:, 0], hs_wr, 50000, BIG_ACC),
        (ei_pu, as_pu[:, 0], ad_pu[:, 0], hs_pu, 5000, SMALL_ACC),
        (ei_rw, as_rw[:, 0], ad_rw[:, 0], hs_rw, 50000, BIG_ACC),
        (ei_rp, as_rp[:, 0], ad_rp[:, 0], hs_rp, 5000, SMALL_ACC),
    ])
    (n_wr, d_wr), (n_pu, d_pu), (n_rw, d_rw), (n_rp, d_rp) = parts

    bl = b_lin.reshape(1, H)
    o_a = _post_one(n_rw, d_rw.T, b_rw.reshape(1, H), W_lin, bl)[:N_AUTHOR]
    o_u = _post_one(n_pu, d_pu.T, b_pu.reshape(1, H), W_lin, bl)[:N_UNIT]
    o_p_head = _post_paper(n_wr, d_wr.T, b_wr.reshape(1, H),
                           n_rp, d_rp.T, b_rp.reshape(1, H),
                           W_lin, bl)[:50000]
    tail = jnp.maximum(0.5 * (b_wr + b_rp), 0.0) @ W_lin + b_lin
    o_p = jnp.concatenate(
        [o_p_head, jnp.broadcast_to(tail, (N_PAPER - 50000, H))])
    return (o_a, o_p, o_u)


# speedup vs baseline: 1.0333x; 1.0333x over previous
import functools

import jax
import jax.numpy as jnp
from jax import lax
from jax.experimental import pallas as pl
from jax.experimental.pallas import tpu as pltpu
from jax.experimental.pallas import tpu_sc as plsc

N_AUTHOR, N_PAPER, N_UNIT = 50000, 100000, 5000
D_IN, H = 128, 32
NEG = -1e30

NC, NS, LANES = 2, 16, 16
NW = NC * NS
CB = 128

BIG_ACC = 50176
SMALL_ACC = 6144


def _src_proj(x, W, A):
    N = x.shape[0]
    R = 1000

    def body(x_ref, w_ref, a_ref, hs_ref, as_ref):
        hs = jnp.dot(x_ref[...], w_ref[...],
                     preferred_element_type=jnp.float32)
        hs_ref[...] = hs
        as_ref[...] = jnp.sum(hs * a_ref[...], axis=1, keepdims=True)

    return pl.pallas_call(
        body,
        grid=(N // R,),
        in_specs=[
            pl.BlockSpec((R, D_IN), lambda i: (i, 0)),
            pl.BlockSpec((D_IN, H), lambda i: (0, 0)),
            pl.BlockSpec((1, H), lambda i: (0, 0)),
        ],
        out_specs=[
            pl.BlockSpec((R, H), lambda i: (i, 0)),
            pl.BlockSpec((R, 1), lambda i: (i, 0)),
        ],
        out_shape=[
            jax.ShapeDtypeStruct((N, H), jnp.float32),
            jax.ShapeDtypeStruct((N, 1), jnp.float32),
        ],
    )(x, W, A)


def _dst_proj(x, wv):
    N = x.shape[0]
    R = 1000

    def body(x_ref, w_ref, ad_ref):
        ad_ref[...] = jnp.dot(x_ref[...], w_ref[...],
                              preferred_element_type=jnp.float32)

    return pl.pallas_call(
        body,
        grid=(N // R,),
        in_specs=[
            pl.BlockSpec((R, D_IN), lambda i: (i, 0)),
            pl.BlockSpec((D_IN, 1), lambda i: (0, 0)),
        ],
        out_specs=pl.BlockSpec((R, 1), lambda i: (i, 0)),
        out_shape=jax.ShapeDtypeStruct((N, 1), jnp.float32),
    )(x, wv)


NBUF = 2


@functools.lru_cache(maxsize=None)
def _mega_kernel(steps4):
    mesh = plsc.VectorSubcoreMesh(core_axis_name="c", subcore_axis_name="s")
    big_chunks = BIG_ACC // CB
    sml_chunks = SMALL_ACC // CB
    big_iters = -(-big_chunks // NS)
    sml_iters = -(-sml_chunks // NS)

    def body(*args):
        srcs, dsts = args[0:4], args[4:8]
        ass, ads, hss = args[8:12], args[12:16], args[16:20]
        m_h = args[20]
        nums, dens = args[21:25], args[25:29]
        scr = args[29:]
        si = scr[0:NBUF]
        di = scr[NBUF:2 * NBUF]
        asv = scr[2 * NBUF:3 * NBUF]
        adv = scr[3 * NBUF:4 * NBUF]
        exv = scr[4 * NBUF:5 * NBUF]
        rows = scr[5 * NBUF:6 * NBUF]
        (zrow, zden, mv, num_big, den_big,
         num_sml, den_sml) = scr[6 * NBUF:6 * NBUF + 7]
        sems = scr[6 * NBUF + 7:]
        ga = sems[0:NBUF]
        gd = sems[NBUF:2 * NBUF]
        gr = sems[2 * NBUF:3 * NBUF]
        sd = sems[3 * NBUF:4 * NBUF]
        sn = sems[4 * NBUF:5 * NBUF]
        cid = lax.axis_index("c")
        sid = lax.axis_index("s")
        wid = sid * NC + cid

        z16 = jnp.zeros((LANES,), jnp.float32)

        def zb(i, _):
            zrow[i, pl.ds(0, LANES)] = z16
            zrow[i, pl.ds(LANES, LANES)] = z16
            return _

        lax.fori_loop(0, CB, zb, None)
        for j in range(CB // LANES):
            zden[pl.ds(j * LANES, LANES)] = z16
        pltpu.sync_copy(m_h, mv)

        def _chunk_loop(iters, total, fn):
            def it(k, _):
                ch = sid + k * NS

                @pl.when(ch < total)
                def _do():
                    fn(ch * CB)

                return _

            lax.fori_loop(0, iters, it, None)

        def zero_accs():
            def z1(r):
                pltpu.sync_copy(zrow, num_big.at[pl.ds(r, CB)])
                pltpu.sync_copy(zden, den_big.at[pl.ds(r, CB)])

            def z2(r):
                pltpu.sync_copy(zrow, num_sml.at[pl.ds(r, CB)])
                pltpu.sync_copy(zden, den_sml.at[pl.ds(r, CB)])

            _chunk_loop(big_iters, big_chunks, z1)
            _chunk_loop(sml_iters, sml_chunks, z2)

        def copy_out(num_sh, den_sh, num_o, den_o, iters, total):
            def co(r):
                pltpu.sync_copy(num_sh.at[pl.ds(r, CB)],
                                num_o.at[cid, pl.ds(r, CB)])
                pltpu.sync_copy(den_sh.at[pl.ds(r, CB)],
                                den_o.at[cid, pl.ds(r, CB)])

            _chunk_loop(iters, total, co)

        def edge_phase(rel, num_sh, den_sh):
            steps = steps4[rel]
            src_h, dst_h = srcs[rel], dsts[rel]
            as_h, ad_h, hs_h = ass[rel], ads[rel], hss[rel]
            mvec = mv[pl.ds(rel * LANES, LANES)]
            wbase = wid * (steps * CB)

            def gathers_start(b, base):
                pltpu.sync_copy(src_h.at[pl.ds(base, CB)], si[b])
                pltpu.sync_copy(dst_h.at[pl.ds(base, CB)], di[b])
                pltpu.async_copy(as_h.at[si[b]], asv[b], ga[b])
                pltpu.async_copy(ad_h.at[di[b]], adv[b], gd[b])
                pltpu.async_copy(hs_h.at[si[b]], rows[b], gr[b])

            def gathers_wait(b):
                pltpu.make_async_copy(as_h.at[si[b]], asv[b], ga[b]).wait()
                pltpu.make_async_copy(ad_h.at[di[b]], adv[b], gd[b]).wait()
                pltpu.make_async_copy(hs_h.at[si[b]], rows[b], gr[b]).wait()

            def scatters_start(b):
                pltpu.async_copy(exv[b], den_sh.at[di[b]], sd[b], add=True)
                pltpu.async_copy(rows[b], num_sh.at[di[b]], sn[b], add=True)

            def scatters_wait(b):
                pltpu.make_async_copy(exv[b], den_sh.at[di[b]], sd[b]).wait()
                pltpu.make_async_copy(rows[b], num_sh.at[di[b]], sn[b]).wait()

            def compute(b):
                for j in range(CB // LANES):
                    a16 = asv[b][pl.ds(j * LANES, LANES)]
                    d16 = adv[b][pl.ds(j * LANES, LANES)]
                    t = a16 + d16
                    e = jnp.maximum(t, 0.2 * t)
                    ex = jnp.exp(e - mvec)
                    exv[b][pl.ds(j * LANES, LANES)] = ex
                    for i in range(LANES):
                        r = j * LANES + i
                        s = ex[i]
                        rows[b][r, pl.ds(0, LANES)] = (
                            rows[b][r, pl.ds(0, LANES)] * s)
                        rows[b][r, pl.ds(LANES, LANES)] = (
                            rows[b][r, pl.ds(LANES, LANES)] * s)

            for b in range(NBUF - 1):
                gathers_start(b, wbase + b * CB)

            def group(p, _):
                for b in range(NBUF):
                    s = NBUF * p + b
                    nxt = (NBUF - 1 + b) % NBUF

                    @pl.when(s + NBUF - 1 < steps)
                    def _issue():
                        @pl.when(s >= 1)
                        def _drain():
                            scatters_wait(nxt)

                        gathers_start(nxt, wbase + (s + NBUF - 1) * CB)

                    gathers_wait(b)
                    compute(b)
                    scatters_start(b)
                return _

            lax.fori_loop(0, steps // NBUF, group, None)
            for b in range(NBUF):
                scatters_wait(b)

        zero_accs()
        plsc.subcore_barrier()
        edge_phase(0, num_big, den_big)
        edge_phase(1, num_sml, den_sml)
        plsc.subcore_barrier()
        copy_out(num_big, den_big, nums[0], dens[0], big_iters, big_chunks)
        copy_out(num_sml, den_sml, nums[1], dens[1], sml_iters, sml_chunks)
        zero_accs()
        plsc.subcore_barrier()
        edge_phase(2, num_big, den_big)
        edge_phase(3, num_sml, den_sml)
        plsc.subcore_barrier()
        copy_out(num_big, den_big, nums[2], dens[2], big_iters, big_chunks)
        copy_out(num_sml, den_sml, nums[3], dens[3], sml_iters, sml_chunks)

    accs = (BIG_ACC, SMALL_ACC, BIG_ACC, SMALL_ACC)
    return pl.kernel(
        body,
        compiler_params=pltpu.CompilerParams(use_tc_tiling_on_sc=False),
        out_type=(
            [jax.ShapeDtypeStruct((NC, a, H), jnp.float32) for a in accs]
            + [jax.ShapeDtypeStruct((NC, a), jnp.float32) for a in accs]
        ),
        mesh=mesh,
        scratch_types=(
            [pltpu.VMEM((CB,), jnp.int32)] * (2 * NBUF)
            + [pltpu.VMEM((CB,), jnp.float32)] * (3 * NBUF)
            + [pltpu.VMEM((CB, H), jnp.float32)] * NBUF
            + [pltpu.VMEM((CB, H), jnp.float32),
               pltpu.VMEM((CB,), jnp.float32),
               pltpu.VMEM((4 * LANES,), jnp.float32),
               pltpu.VMEM_SHARED((BIG_ACC, H), jnp.float32),
               pltpu.VMEM_SHARED((BIG_ACC,), jnp.float32),
               pltpu.VMEM_SHARED((SMALL_ACC, H), jnp.float32),
               pltpu.VMEM_SHARED((SMALL_ACC,), jnp.float32)]
            + [pltpu.SemaphoreType.DMA] * (5 * NBUF)
        ),
    )


def _pad_relation(ei, a_d, n_dst_real, n_acc):
    E = ei.shape[1]
    quant = NBUF * NW * CB
    e_pad = ((E + quant - 1) // quant) * quant
    pad = e_pad - E
    src = jnp.concatenate([ei[0].astype(jnp.int32),
                           jnp.zeros((pad,), jnp.int32)])
    dst = jnp.concatenate([ei[1].astype(jnp.int32),
                           jnp.full((pad,), n_dst_real, jnp.int32)])
    ad_pad = jnp.concatenate(
        [a_d, jnp.full((n_acc - a_d.shape[0],), NEG, jnp.float32)])
    return src, dst, ad_pad, e_pad // (NW * CB)


def _run_all(rels):
    srcs, dsts, ads, steps4, ms = [], [], [], [], []
    for ei, a_s, a_d, hs, n_dst_real, n_acc in rels:
        src, dst, ad_pad, steps = _pad_relation(ei, a_d, n_dst_real, n_acc)
        srcs.append(src)
        dsts.append(dst)
        ads.append(ad_pad)
        steps4.append(steps)
        t = jnp.max(a_s) + jnp.max(a_d)
        ms.append(jnp.full((LANES,), jnp.maximum(t, 0.2 * t), jnp.float32))
    m_all = jnp.concatenate(ms)
    k = _mega_kernel(tuple(steps4))
    outs = k(*srcs, *dsts, *[r[1] for r in rels], *ads,
             *[r[3] for r in rels], m_all)
    return list(zip(outs[0:4], outs[4:8]))


def _post_one(num, den_t, b, W_lin, b_lin):
    N = num.shape[1]
    R = 512

    def body(n_ref, d_ref, b_ref, wl_ref, bl_ref, o_ref):
        nm = n_ref[0] + n_ref[1]
        dn = d_ref[..., 0:1] + d_ref[..., 1:2]
        o = nm / (dn + 1e-16) + b_ref[...]
        o_ref[...] = jnp.dot(jnp.maximum(o, 0.0), wl_ref[...],
                             preferred_element_type=jnp.float32) + bl_ref[...]

    return pl.pallas_call(
        body,
        grid=(N // R,),
        in_specs=[
            pl.BlockSpec((NC, R, H), lambda i: (0, i, 0)),
            pl.BlockSpec((R, NC), lambda i: (i, 0)),
            pl.BlockSpec((1, H), lambda i: (0, 0)),
            pl.BlockSpec((H, H), lambda i: (0, 0)),
            pl.BlockSpec((1, H), lambda i: (0, 0)),
        ],
        out_specs=pl.BlockSpec((R, H), lambda i: (i, 0)),
        out_shape=jax.ShapeDtypeStruct((N, H), jnp.float32),
    )(num, den_t, b, W_lin, b_lin)


def _post_paper(num1, den1_t, b1, num2, den2_t, b2, W_lin, b_lin):
    R = 512
    last2 = SMALL_ACC // R - 1

    def body(n1, d1, bb1, n2, d2, bb2, wl, bl, o_ref):
        o1 = (n1[0] + n1[1]) / (d1[..., 0:1] + d1[..., 1:2] + 1e-16) + bb1[...]
        o2 = (n2[0] + n2[1]) / (d2[..., 0:1] + d2[..., 1:2] + 1e-16) + bb2[...]
        o = 0.5 * (o1 + o2)
        o_ref[...] = jnp.dot(jnp.maximum(o, 0.0), wl[...],
                             preferred_element_type=jnp.float32) + bl[...]

    return pl.pallas_call(
        body,
        grid=(BIG_ACC // R,),
        in_specs=[
            pl.BlockSpec((NC, R, H), lambda i: (0, i, 0)),
            pl.BlockSpec((R, NC), lambda i: (i, 0)),
            pl.BlockSpec((1, H), lambda i: (0, 0)),
            pl.BlockSpec((NC, R, H), lambda i: (0, jnp.minimum(i, last2), 0)),
            pl.BlockSpec((R, NC), lambda i: (jnp.minimum(i, last2), 0)),
            pl.BlockSpec((1, H), lambda i: (0, 0)),
            pl.BlockSpec((H, H), lambda i: (0, 0)),
            pl.BlockSpec((1, H), lambda i: (0, 0)),
        ],
        out_specs=pl.BlockSpec((R, H), lambda i: (i, 0)),
        out_shape=jax.ShapeDtypeStruct((BIG_ACC, H), jnp.float32),
    )(num1, den1_t, b1, num2, den2_t, b2, W_lin, b_lin)


def kernel(x_author, x_paper, x_unit,
           Ws_wr, Wd_wr, As_wr, Ad_wr, b_wr,
           Ws_pu, Wd_pu, As_pu, Ad_pu, b_pu,
           Ws_rw, Wd_rw, As_rw, Ad_rw, b_rw,
           Ws_rp, Wd_rp, As_rp, Ad_rp, b_rp,
           W_lin, b_lin,
           ei_wr, ei_pu, ei_rw, ei_rp):
    xp50 = x_paper[:50000]
    xp5 = x_paper[:5000]

    hs_wr, as_wr = _src_proj(x_author, Ws_wr, As_wr.reshape(1, H))
    hs_rw, as_rw = _src_proj(xp50, Ws_rw, As_rw.reshape(1, H))
    hs_pu, as_pu = _src_proj(xp5, Ws_pu, As_pu.reshape(1, H))
    hs_rp, as_rp = _src_proj(x_unit, Ws_rp, As_rp.reshape(1, H))
    ad_wr = _dst_proj(xp50, (Wd_wr @ Ad_wr).reshape(D_IN, 1))
    ad_rw = _dst_proj(x_author, (Wd_rw @ Ad_rw).reshape(D_IN, 1))
    ad_pu = _dst_proj(x_unit, (Wd_pu @ Ad_pu).reshape(D_IN, 1))
    ad_rp = _dst_proj(xp5, (Wd_rp @ Ad_rp).reshape(D_IN, 1))

    parts = _run_all([
        (ei_wr, as_wr[:, 0], ad_wr[:, 0], hs_wr, 50000, BIG_ACC),
        (ei_pu, as_pu[:, 0], ad_pu[:, 0], hs_pu, 5000, SMALL_ACC),
        (ei_rw, as_rw[:, 0], ad_rw[:, 0], hs_rw, 50000, BIG_ACC),
        (ei_rp, as_rp[:, 0], ad_rp[:, 0], hs_rp, 5000, SMALL_ACC),
    ])
    (n_wr, d_wr), (n_pu, d_pu), (n_rw, d_rw), (n_rp, d_rp) = parts

    bl = b_lin.reshape(1, H)
    o_a = _post_one(n_rw, d_rw.T, b_rw.reshape(1, H), W_lin, bl)[:N_AUTHOR]
    o_u = _post_one(n_pu, d_pu.T, b_pu.reshape(1, H), W_lin, bl)[:N_UNIT]
    o_p_head = _post_paper(n_wr, d_wr.T, b_wr.reshape(1, H),
                           n_rp, d_rp.T, b_rp.reshape(1, H),
                           W_lin, bl)[:50000]
    tail = jnp.maximum(0.5 * (b_wr + b_rp), 0.0) @ W_lin + b_lin
    o_p = jnp.concatenate(
        [o_p_head, jnp.broadcast_to(tail, (N_PAPER - 50000, H))])
    return (o_a, o_p, o_u)


# speedup vs baseline: 1.1824x; 1.1443x over previous
import functools

import jax
import jax.numpy as jnp
from jax import lax
from jax.experimental import pallas as pl
from jax.experimental.pallas import tpu as pltpu
from jax.experimental.pallas import tpu_sc as plsc

N_AUTHOR, N_PAPER, N_UNIT = 50000, 100000, 5000
D_IN, H = 128, 32
NEG = -1e30

NC, NS, LANES = 2, 16, 16
NW = NC * NS
CB = 128

BIG_ACC = 50176
SMALL_ACC = 6144


def _src_proj(x, W, A):
    N = x.shape[0]
    R = 1000

    def body(x_ref, w_ref, a_ref, hs_ref, as_ref):
        hs = jnp.dot(x_ref[...], w_ref[...],
                     preferred_element_type=jnp.float32)
        hs_ref[...] = hs
        as_ref[...] = jnp.sum(hs * a_ref[...], axis=1, keepdims=True)

    return pl.pallas_call(
        body,
        grid=(N // R,),
        in_specs=[
            pl.BlockSpec((R, D_IN), lambda i: (i, 0)),
            pl.BlockSpec((D_IN, H), lambda i: (0, 0)),
            pl.BlockSpec((1, H), lambda i: (0, 0)),
        ],
        out_specs=[
            pl.BlockSpec((R, H), lambda i: (i, 0)),
            pl.BlockSpec((R, 1), lambda i: (i, 0)),
        ],
        out_shape=[
            jax.ShapeDtypeStruct((N, H), jnp.float32),
            jax.ShapeDtypeStruct((N, 1), jnp.float32),
        ],
    )(x, W, A)


def _dst_proj(x, wv):
    N = x.shape[0]
    R = 1000

    def body(x_ref, w_ref, ad_ref):
        ad_ref[...] = jnp.dot(x_ref[...], w_ref[...],
                              preferred_element_type=jnp.float32)

    return pl.pallas_call(
        body,
        grid=(N // R,),
        in_specs=[
            pl.BlockSpec((R, D_IN), lambda i: (i, 0)),
            pl.BlockSpec((D_IN, 1), lambda i: (0, 0)),
        ],
        out_specs=pl.BlockSpec((R, 1), lambda i: (i, 0)),
        out_shape=jax.ShapeDtypeStruct((N, 1), jnp.float32),
    )(x, wv)


NBUF = 2


@functools.lru_cache(maxsize=None)
def _edge_kernel(e_pad, n_src, n_acc):
    steps = e_pad // (NW * CB)
    assert steps % NBUF == 0 and steps >= NBUF
    n_chunks = n_acc // CB
    n_iters = -(-n_chunks // NS)
    mesh = plsc.VectorSubcoreMesh(core_axis_name="c", subcore_axis_name="s")

    def body(src_h, dst_h, as_h, ad_h, hs_h, m_h,
             num_o, den_o, *scr):
        si = scr[0:NBUF]
        di = scr[NBUF:2 * NBUF]
        asv = scr[2 * NBUF:3 * NBUF]
        adv = scr[3 * NBUF:4 * NBUF]
        exv = scr[4 * NBUF:5 * NBUF]
        rows = scr[5 * NBUF:6 * NBUF]
        zrow, zden, mv, num_sh, den_sh = scr[6 * NBUF:6 * NBUF + 5]
        sems = scr[6 * NBUF + 5:]
        ga = sems[0:NBUF]
        gd = sems[NBUF:2 * NBUF]
        gr = sems[2 * NBUF:3 * NBUF]
        sd = sems[3 * NBUF:4 * NBUF]
        sn = sems[4 * NBUF:5 * NBUF]
        cid = lax.axis_index("c")
        sid = lax.axis_index("s")
        wid = sid * NC + cid

        z16 = jnp.zeros((LANES,), jnp.float32)

        def zb(i, _):
            zrow[i, pl.ds(0, LANES)] = z16
            zrow[i, pl.ds(LANES, LANES)] = z16
            return _

        lax.fori_loop(0, CB, zb, None)
        for j in range(CB // LANES):
            zden[pl.ds(j * LANES, LANES)] = z16
        pltpu.sync_copy(m_h, mv)

        def _chunk_loop(fn):
            def it(k, _):
                ch = sid + k * NS

                @pl.when(ch < n_chunks)
                def _do():
                    fn(ch * CB)

                return _

            lax.fori_loop(0, n_iters, it, None)

        def zero1(r):
            pltpu.sync_copy(zrow, num_sh.at[pl.ds(r, CB)])
            pltpu.sync_copy(zden, den_sh.at[pl.ds(r, CB)])

        _chunk_loop(zero1)
        plsc.subcore_barrier()

        mvec = mv[...]
        wbase = wid * (steps * CB)

        def gathers_start(b, base):
            pltpu.sync_copy(src_h.at[pl.ds(base, CB)], si[b])
            pltpu.sync_copy(dst_h.at[pl.ds(base, CB)], di[b])
            pltpu.async_copy(as_h.at[si[b]], asv[b], ga[b])
            pltpu.async_copy(ad_h.at[di[b]], adv[b], gd[b])
            pltpu.async_copy(hs_h.at[si[b]], rows[b], gr[b])

        def gathers_wait(b):
            pltpu.make_async_copy(as_h.at[si[b]], asv[b], ga[b]).wait()
            pltpu.make_async_copy(ad_h.at[di[b]], adv[b], gd[b]).wait()
            pltpu.make_async_copy(hs_h.at[si[b]], rows[b], gr[b]).wait()

        def scatters_start(b):
            pltpu.async_copy(exv[b], den_sh.at[di[b]], sd[b], add=True)
            pltpu.async_copy(rows[b], num_sh.at[di[b]], sn[b], add=True)

        def scatters_wait(b):
            pltpu.make_async_copy(exv[b], den_sh.at[di[b]], sd[b]).wait()
            pltpu.make_async_copy(rows[b], num_sh.at[di[b]], sn[b]).wait()

        def compute(b):
            for j in range(CB // LANES):
                a16 = asv[b][pl.ds(j * LANES, LANES)]
                d16 = adv[b][pl.ds(j * LANES, LANES)]
                t = a16 + d16
                e = jnp.maximum(t, 0.2 * t)
                ex = jnp.exp(e - mvec)
                exv[b][pl.ds(j * LANES, LANES)] = ex
                for i in range(LANES):
                    r = j * LANES + i
                    s = ex[i]
                    rows[b][r, pl.ds(0, LANES)] = (
                        rows[b][r, pl.ds(0, LANES)] * s)
                    rows[b][r, pl.ds(LANES, LANES)] = (
                        rows[b][r, pl.ds(LANES, LANES)] * s)

        for b in range(NBUF - 1):
            gathers_start(b, wbase + b * CB)

        def group(p, _):
            for b in range(NBUF):
                s = NBUF * p + b
                nxt = (NBUF - 1 + b) % NBUF

                @pl.when(s + NBUF - 1 < steps)
                def _issue():
                    @pl.when(s >= 1)
                    def _drain():
                        scatters_wait(nxt)

                    gathers_start(nxt, wbase + (s + NBUF - 1) * CB)

                gathers_wait(b)
                compute(b)
                scatters_start(b)
            return _

        lax.fori_loop(0, steps // NBUF, group, None)
        for b in range(NBUF):
            scatters_wait(b)
        plsc.subcore_barrier()

        def out1(r):
            pltpu.sync_copy(num_sh.at[pl.ds(r, CB)],
                            num_o.at[cid, pl.ds(r, CB)])
            pltpu.sync_copy(den_sh.at[pl.ds(r, CB)],
                            den_o.at[cid, pl.ds(r, CB)])

        _chunk_loop(out1)

    return pl.kernel(
        body,
        compiler_params=pltpu.CompilerParams(use_tc_tiling_on_sc=False),
        out_type=[
            jax.ShapeDtypeStruct((NC, n_acc, H), jnp.float32),
            jax.ShapeDtypeStruct((NC, n_acc), jnp.float32),
        ],
        mesh=mesh,
        scratch_types=(
            [pltpu.VMEM((CB,), jnp.int32)] * (2 * NBUF)
            + [pltpu.VMEM((CB,), jnp.float32)] * (3 * NBUF)
            + [pltpu.VMEM((CB, H), jnp.float32)] * NBUF
            + [pltpu.VMEM((CB, H), jnp.float32),
               pltpu.VMEM((CB,), jnp.float32),
               pltpu.VMEM((LANES,), jnp.float32),
               pltpu.VMEM_SHARED((n_acc, H), jnp.float32),
               pltpu.VMEM_SHARED((n_acc,), jnp.float32)]
            + [pltpu.SemaphoreType.DMA] * (5 * NBUF)
        ),
    )


def _run_relation(ei, a_s, a_d, hs, n_dst_real, n_acc):
    E = ei.shape[1]
    quant = NBUF * NW * CB
    e_pad = ((E + quant - 1) // quant) * quant
    pad = e_pad - E
    src = jnp.concatenate([ei[0].astype(jnp.int32),
                           jnp.zeros((pad,), jnp.int32)])
    dst = jnp.concatenate([ei[1].astype(jnp.int32),
                           jnp.full((pad,), n_dst_real, jnp.int32)])
    ad_pad = jnp.concatenate(
        [a_d, jnp.full((n_acc - a_d.shape[0],), NEG, jnp.float32)])
    t = jnp.max(a_s) + jnp.max(a_d)
    m = jnp.maximum(t, 0.2 * t)
    m_arr = jnp.full((LANES,), m, jnp.float32)
    k = _edge_kernel(e_pad, hs.shape[0], n_acc)
    return k(src, dst, a_s, ad_pad, hs, m_arr)


def _post_one(num, den_t, b, W_lin, b_lin):
    N = num.shape[1]
    R = 512

    def body(n_ref, d_ref, b_ref, wl_ref, bl_ref, o_ref):
        nm = n_ref[0] + n_ref[1]
        dn = d_ref[..., 0:1] + d_ref[..., 1:2]
        o = nm / (dn + 1e-16) + b_ref[...]
        o_ref[...] = jnp.dot(jnp.maximum(o, 0.0), wl_ref[...],
                             preferred_element_type=jnp.float32) + bl_ref[...]

    return pl.pallas_call(
        body,
        grid=(N // R,),
        in_specs=[
            pl.BlockSpec((NC, R, H), lambda i: (0, i, 0)),
            pl.BlockSpec((R, NC), lambda i: (i, 0)),
            pl.BlockSpec((1, H), lambda i: (0, 0)),
            pl.BlockSpec((H, H), lambda i: (0, 0)),
            pl.BlockSpec((1, H), lambda i: (0, 0)),
        ],
        out_specs=pl.BlockSpec((R, H), lambda i: (i, 0)),
        out_shape=jax.ShapeDtypeStruct((N, H), jnp.float32),
    )(num, den_t, b, W_lin, b_lin)


def _post_paper(num1, den1_t, b1, num2, den2_t, b2, W_lin, b_lin):
    R = 512
    last2 = SMALL_ACC // R - 1

    def body(n1, d1, bb1, n2, d2, bb2, wl, bl, o_ref):
        o1 = (n1[0] + n1[1]) / (d1[..., 0:1] + d1[..., 1:2] + 1e-16) + bb1[...]
        o2 = (n2[0] + n2[1]) / (d2[..., 0:1] + d2[..., 1:2] + 1e-16) + bb2[...]
        o = 0.5 * (o1 + o2)
        o_ref[...] = jnp.dot(jnp.maximum(o, 0.0), wl[...],
                             preferred_element_type=jnp.float32) + bl[...]

    return pl.pallas_call(
        body,
        grid=(BIG_ACC // R,),
        in_specs=[
            pl.BlockSpec((NC, R, H), lambda i: (0, i, 0)),
            pl.BlockSpec((R, NC), lambda i: (i, 0)),
            pl.BlockSpec((1, H), lambda i: (0, 0)),
            pl.BlockSpec((NC, R, H), lambda i: (0, jnp.minimum(i, last2), 0)),
            pl.BlockSpec((R, NC), lambda i: (jnp.minimum(i, last2), 0)),
            pl.BlockSpec((1, H), lambda i: (0, 0)),
            pl.BlockSpec((H, H), lambda i: (0, 0)),
            pl.BlockSpec((1, H), lambda i: (0, 0)),
        ],
        out_specs=pl.BlockSpec((R, H), lambda i: (i, 0)),
        out_shape=jax.ShapeDtypeStruct((BIG_ACC, H), jnp.float32),
    )(num1, den1_t, b1, num2, den2_t, b2, W_lin, b_lin)


def kernel(x_author, x_paper, x_unit,
           Ws_wr, Wd_wr, As_wr, Ad_wr, b_wr,
           Ws_pu, Wd_pu, As_pu, Ad_pu, b_pu,
           Ws_rw, Wd_rw, As_rw, Ad_rw, b_rw,
           Ws_rp, Wd_rp, As_rp, Ad_rp, b_rp,
           W_lin, b_lin,
           ei_wr, ei_pu, ei_rw, ei_rp):
    xp50 = x_paper[:50000]
    xp5 = x_paper[:5000]

    hs_wr, as_wr = _src_proj(x_author, Ws_wr, As_wr.reshape(1, H))
    hs_rw, as_rw = _src_proj(xp50, Ws_rw, As_rw.reshape(1, H))
    hs_pu, as_pu = _src_proj(xp5, Ws_pu, As_pu.reshape(1, H))
    hs_rp, as_rp = _src_proj(x_unit, Ws_rp, As_rp.reshape(1, H))
    ad_wr = _dst_proj(xp50, (Wd_wr @ Ad_wr).reshape(D_IN, 1))
    ad_rw = _dst_proj(x_author, (Wd_rw @ Ad_rw).reshape(D_IN, 1))
    ad_pu = _dst_proj(x_unit, (Wd_pu @ Ad_pu).reshape(D_IN, 1))
    ad_rp = _dst_proj(xp5, (Wd_rp @ Ad_rp).reshape(D_IN, 1))

    n_wr, d_wr = _run_relation(ei_wr, as_wr[:, 0], ad_wr[:, 0], hs_wr,
                               50000, BIG_ACC)
    n_pu, d_pu = _run_relation(ei_pu, as_pu[:, 0], ad_pu[:, 0], hs_pu,
                               5000, SMALL_ACC)
    n_rw, d_rw = _run_relation(ei_rw, as_rw[:, 0], ad_rw[:, 0], hs_rw,
                               50000, BIG_ACC)
    n_rp, d_rp = _run_relation(ei_rp, as_rp[:, 0], ad_rp[:, 0], hs_rp,
                               5000, SMALL_ACC)

    bl = b_lin.reshape(1, H)
    o_a = _post_one(n_rw, d_rw.T, b_rw.reshape(1, H), W_lin, bl)[:N_AUTHOR]
    o_u = _post_one(n_pu, d_pu.T, b_pu.reshape(1, H), W_lin, bl)[:N_UNIT]
    o_p_head = _post_paper(n_wr, d_wr.T, b_wr.reshape(1, H),
                           n_rp, d_rp.T, b_rp.reshape(1, H),
                           W_lin, bl)[:50000]
    tail = jnp.maximum(0.5 * (b_wr + b_rp), 0.0) @ W_lin + b_lin
    o_p = jnp.concatenate(
        [o_p_head, jnp.broadcast_to(tail, (N_PAPER - 50000, H))])
    return (o_a, o_p, o_u)


# speedup vs baseline: 1.2633x; 1.0684x over previous
import functools

import jax
import jax.numpy as jnp
from jax import lax
from jax.experimental import pallas as pl
from jax.experimental.pallas import tpu as pltpu
from jax.experimental.pallas import tpu_sc as plsc

N_AUTHOR, N_PAPER, N_UNIT = 50000, 100000, 5000
D_IN, H = 128, 32
NEG = -1e30

NC, NS, LANES = 2, 16, 16
NW = NC * NS
CB = 128

BIG_ACC = 51200
SMALL_ACC = 6144


def _src_proj(x, W, A):
    N = x.shape[0]
    R = 1000

    def body(x_ref, w_ref, a_ref, hs_ref, as_ref):
        hs = jnp.dot(x_ref[...], w_ref[...],
                     preferred_element_type=jnp.float32)
        hs_ref[...] = hs
        as_ref[...] = jnp.sum(hs * a_ref[...], axis=1, keepdims=True)

    return pl.pallas_call(
        body,
        grid=(N // R,),
        in_specs=[
            pl.BlockSpec((R, D_IN), lambda i: (i, 0)),
            pl.BlockSpec((D_IN, H), lambda i: (0, 0)),
            pl.BlockSpec((1, H), lambda i: (0, 0)),
        ],
        out_specs=[
            pl.BlockSpec((R, H), lambda i: (i, 0)),
            pl.BlockSpec((R, 1), lambda i: (i, 0)),
        ],
        out_shape=[
            jax.ShapeDtypeStruct((N, H), jnp.float32),
            jax.ShapeDtypeStruct((N, 1), jnp.float32),
        ],
    )(x, W, A)


def _dst_proj(x, wv):
    N = x.shape[0]
    R = 1000

    def body(x_ref, w_ref, ad_ref):
        ad_ref[...] = jnp.dot(x_ref[...], w_ref[...],
                              preferred_element_type=jnp.float32)

    return pl.pallas_call(
        body,
        grid=(N // R,),
        in_specs=[
            pl.BlockSpec((R, D_IN), lambda i: (i, 0)),
            pl.BlockSpec((D_IN, 1), lambda i: (0, 0)),
        ],
        out_specs=pl.BlockSpec((R, 1), lambda i: (i, 0)),
        out_shape=jax.ShapeDtypeStruct((N, 1), jnp.float32),
    )(x, wv)


NBUF = 2


@functools.lru_cache(maxsize=None)
def _edge_kernel(e_pad, n_src, n_acc):
    steps = e_pad // (NW * CB)
    assert steps % NBUF == 0 and steps >= NBUF
    n_chunks = n_acc // CB
    n_iters = -(-n_chunks // NS)
    mesh = plsc.VectorSubcoreMesh(core_axis_name="c", subcore_axis_name="s")

    def body(src_h, dst_h, as_h, ad_h, hs_h, m_h,
             num_o, den_o, *scr):
        si = scr[0:NBUF]
        di = scr[NBUF:2 * NBUF]
        asv = scr[2 * NBUF:3 * NBUF]
        adv = scr[3 * NBUF:4 * NBUF]
        exv = scr[4 * NBUF:5 * NBUF]
        rows = scr[5 * NBUF:6 * NBUF]
        zrow, zden, mv, num_sh, den_sh = scr[6 * NBUF:6 * NBUF + 5]
        sems = scr[6 * NBUF + 5:]
        ga = sems[0:NBUF]
        gd = sems[NBUF:2 * NBUF]
        gr = sems[2 * NBUF:3 * NBUF]
        sd = sems[3 * NBUF:4 * NBUF]
        sn = sems[4 * NBUF:5 * NBUF]
        cid = lax.axis_index("c")
        sid = lax.axis_index("s")
        wid = sid * NC + cid

        z16 = jnp.zeros((LANES,), jnp.float32)

        def zb(i, _):
            zrow[i, pl.ds(0, LANES)] = z16
            zrow[i, pl.ds(LANES, LANES)] = z16
            return _

        lax.fori_loop(0, CB, zb, None)
        for j in range(CB // LANES):
            zden[pl.ds(j * LANES, LANES)] = z16
        pltpu.sync_copy(m_h, mv)

        def _chunk_loop(fn):
            def it(k, _):
                ch = sid + k * NS

                @pl.when(ch < n_chunks)
                def _do():
                    fn(ch * CB)

                return _

            lax.fori_loop(0, n_iters, it, None)

        def zero1(r):
            pltpu.sync_copy(zrow, num_sh.at[pl.ds(r, CB)])
            pltpu.sync_copy(zden, den_sh.at[pl.ds(r, CB)])

        _chunk_loop(zero1)
        plsc.subcore_barrier()

        mvec = mv[...]
        wbase = wid * (steps * CB)

        def gathers_start(b, base):
            pltpu.sync_copy(src_h.at[pl.ds(base, CB)], si[b])
            pltpu.sync_copy(dst_h.at[pl.ds(base, CB)], di[b])
            pltpu.async_copy(as_h.at[si[b]], asv[b], ga[b])
            pltpu.async_copy(ad_h.at[di[b]], adv[b], gd[b])
            pltpu.async_copy(hs_h.at[si[b]], rows[b], gr[b])

        def gathers_wait(b):
            pltpu.make_async_copy(as_h.at[si[b]], asv[b], ga[b]).wait()
            pltpu.make_async_copy(ad_h.at[di[b]], adv[b], gd[b]).wait()
            pltpu.make_async_copy(hs_h.at[si[b]], rows[b], gr[b]).wait()

        def scatters_start(b):
            pltpu.async_copy(exv[b], den_sh.at[di[b]], sd[b], add=True)
            pltpu.async_copy(rows[b], num_sh.at[di[b]], sn[b], add=True)

        def scatters_wait(b):
            pltpu.make_async_copy(exv[b], den_sh.at[di[b]], sd[b]).wait()
            pltpu.make_async_copy(rows[b], num_sh.at[di[b]], sn[b]).wait()

        def compute(b):
            for j in range(CB // LANES):
                a16 = asv[b][pl.ds(j * LANES, LANES)]
                d16 = adv[b][pl.ds(j * LANES, LANES)]
                t = a16 + d16
                e = jnp.maximum(t, 0.2 * t)
                ex = jnp.exp(e - mvec)
                exv[b][pl.ds(j * LANES, LANES)] = ex
                for i in range(LANES):
                    r = j * LANES + i
                    s = ex[i]
                    rows[b][r, pl.ds(0, LANES)] = (
                        rows[b][r, pl.ds(0, LANES)] * s)
                    rows[b][r, pl.ds(LANES, LANES)] = (
                        rows[b][r, pl.ds(LANES, LANES)] * s)

        for b in range(NBUF - 1):
            gathers_start(b, wbase + b * CB)

        def group(p, _):
            for b in range(NBUF):
                s = NBUF * p + b
                nxt = (NBUF - 1 + b) % NBUF

                @pl.when(s + NBUF - 1 < steps)
                def _issue():
                    @pl.when(s >= 1)
                    def _drain():
                        scatters_wait(nxt)

                    gathers_start(nxt, wbase + (s + NBUF - 1) * CB)

                gathers_wait(b)
                compute(b)
                scatters_start(b)
            return _

        lax.fori_loop(0, steps // NBUF, group, None)
        for b in range(NBUF):
            scatters_wait(b)
        plsc.subcore_barrier()

        def out1(r):
            pltpu.sync_copy(num_sh.at[pl.ds(r, CB)],
                            num_o.at[cid, pl.ds(r, CB)])
            pltpu.sync_copy(den_sh.at[pl.ds(r, CB)],
                            den_o.at[cid, pl.ds(r, CB)])

        _chunk_loop(out1)

    return pl.kernel(
        body,
        compiler_params=pltpu.CompilerParams(use_tc_tiling_on_sc=False),
        out_type=[
            jax.ShapeDtypeStruct((NC, n_acc, H), jnp.float32),
            jax.ShapeDtypeStruct((NC, n_acc), jnp.float32),
        ],
        mesh=mesh,
        scratch_types=(
            [pltpu.VMEM((CB,), jnp.int32)] * (2 * NBUF)
            + [pltpu.VMEM((CB,), jnp.float32)] * (3 * NBUF)
            + [pltpu.VMEM((CB, H), jnp.float32)] * NBUF
            + [pltpu.VMEM((CB, H), jnp.float32),
               pltpu.VMEM((CB,), jnp.float32),
               pltpu.VMEM((LANES,), jnp.float32),
               pltpu.VMEM_SHARED((n_acc, H), jnp.float32),
               pltpu.VMEM_SHARED((n_acc,), jnp.float32)]
            + [pltpu.SemaphoreType.DMA] * (5 * NBUF)
        ),
    )


def _run_relation(ei, a_s, a_d, hs, n_dst_real, n_acc):
    E = ei.shape[1]
    quant = NBUF * NW * CB
    e_pad = ((E + quant - 1) // quant) * quant
    pad = e_pad - E
    src = jnp.concatenate([ei[0].astype(jnp.int32),
                           jnp.zeros((pad,), jnp.int32)])
    dst = jnp.concatenate([ei[1].astype(jnp.int32),
                           jnp.full((pad,), n_dst_real, jnp.int32)])
    ad_pad = jnp.concatenate(
        [a_d, jnp.full((n_acc - a_d.shape[0],), NEG, jnp.float32)])
    t = jnp.max(a_s) + jnp.max(a_d)
    m = jnp.maximum(t, 0.2 * t)
    m_arr = jnp.full((LANES,), m, jnp.float32)
    k = _edge_kernel(e_pad, hs.shape[0], n_acc)
    return k(src, dst, a_s, ad_pad, hs, m_arr)


def _post_one(num, den_t, b, W_lin, b_lin):
    N = num.shape[1]
    R = 512

    def body(n_ref, d_ref, b_ref, wl_ref, bl_ref, o_ref):
        nm = n_ref[0] + n_ref[1]
        dn = d_ref[..., 0:1] + d_ref[..., 1:2]
        o = nm / (dn + 1e-16) + b_ref[...]
        o_ref[...] = jnp.dot(jnp.maximum(o, 0.0), wl_ref[...],
                             preferred_element_type=jnp.float32) + bl_ref[...]

    return pl.pallas_call(
        body,
        grid=(N // R,),
        in_specs=[
            pl.BlockSpec((NC, R, H), lambda i: (0, i, 0)),
            pl.BlockSpec((R, NC), lambda i: (i, 0)),
            pl.BlockSpec((1, H), lambda i: (0, 0)),
            pl.BlockSpec((H, H), lambda i: (0, 0)),
            pl.BlockSpec((1, H), lambda i: (0, 0)),
        ],
        out_specs=pl.BlockSpec((R, H), lambda i: (i, 0)),
        out_shape=jax.ShapeDtypeStruct((N, H), jnp.float32),
    )(num, den_t, b, W_lin, b_lin)


def _post_paper(num1, den1_t, b1, num2, den2_t, b2, W_lin, b_lin):
    R = 512
    last2 = SMALL_ACC // R - 1

    def body(n1, d1, bb1, n2, d2, bb2, wl, bl, o_ref):
        o1 = (n1[0] + n1[1]) / (d1[..., 0:1] + d1[..., 1:2] + 1e-16) + bb1[...]
        o2 = (n2[0] + n2[1]) / (d2[..., 0:1] + d2[..., 1:2] + 1e-16) + bb2[...]
        o = 0.5 * (o1 + o2)
        o_ref[...] = jnp.dot(jnp.maximum(o, 0.0), wl[...],
                             preferred_element_type=jnp.float32) + bl[...]

    return pl.pallas_call(
        body,
        grid=(BIG_ACC // R,),
        in_specs=[
            pl.BlockSpec((NC, R, H), lambda i: (0, i, 0)),
            pl.BlockSpec((R, NC), lambda i: (i, 0)),
            pl.BlockSpec((1, H), lambda i: (0, 0)),
            pl.BlockSpec((NC, R, H), lambda i: (0, jnp.minimum(i, last2), 0)),
            pl.BlockSpec((R, NC), lambda i: (jnp.minimum(i, last2), 0)),
            pl.BlockSpec((1, H), lambda i: (0, 0)),
            pl.BlockSpec((H, H), lambda i: (0, 0)),
            pl.BlockSpec((1, H), lambda i: (0, 0)),
        ],
        out_specs=pl.BlockSpec((R, H), lambda i: (i, 0)),
        out_shape=jax.ShapeDtypeStruct((BIG_ACC, H), jnp.float32),
    )(num1, den1_t, b1, num2, den2_t, b2, W_lin, b_lin)


def kernel(x_author, x_paper, x_unit,
           Ws_wr, Wd_wr, As_wr, Ad_wr, b_wr,
           Ws_pu, Wd_pu, As_pu, Ad_pu, b_pu,
           Ws_rw, Wd_rw, As_rw, Ad_rw, b_rw,
           Ws_rp, Wd_rp, As_rp, Ad_rp, b_rp,
           W_lin, b_lin,
           ei_wr, ei_pu, ei_rw, ei_rp):
    xp50 = x_paper[:50000]
    xp5 = x_paper[:5000]

    hs_wr, as_wr = _src_proj(x_author, Ws_wr, As_wr.reshape(1, H))
    hs_rw, as_rw = _src_proj(xp50, Ws_rw, As_rw.reshape(1, H))
    hs_pu, as_pu = _src_proj(xp5, Ws_pu, As_pu.reshape(1, H))
    hs_rp, as_rp = _src_proj(x_unit, Ws_rp, As_rp.reshape(1, H))
    ad_wr = _dst_proj(xp50, (Wd_wr @ Ad_wr).reshape(D_IN, 1))
    ad_rw = _dst_proj(x_author, (Wd_rw @ Ad_rw).reshape(D_IN, 1))
    ad_pu = _dst_proj(x_unit, (Wd_pu @ Ad_pu).reshape(D_IN, 1))
    ad_rp = _dst_proj(xp5, (Wd_rp @ Ad_rp).reshape(D_IN, 1))

    n_wr, d_wr = _run_relation(ei_wr, as_wr[:, 0], ad_wr[:, 0], hs_wr,
                               50000, BIG_ACC)
    n_pu, d_pu = _run_relation(ei_pu, as_pu[:, 0], ad_pu[:, 0], hs_pu,
                               5000, SMALL_ACC)
    n_rw, d_rw = _run_relation(ei_rw, as_rw[:, 0], ad_rw[:, 0], hs_rw,
                               50000, BIG_ACC)
    n_rp, d_rp = _run_relation(ei_rp, as_rp[:, 0], ad_rp[:, 0], hs_rp,
                               5000, SMALL_ACC)

    bl = b_lin.reshape(1, H)
    o_a = _post_one(n_rw, d_rw.T, b_rw.reshape(1, H), W_lin, bl)[:N_AUTHOR]
    o_u = _post_one(n_pu, d_pu.T, b_pu.reshape(1, H), W_lin, bl)[:N_UNIT]
    o_p_head = _post_paper(n_wr, d_wr.T, b_wr.reshape(1, H),
                           n_rp, d_rp.T, b_rp.reshape(1, H),
                           W_lin, bl)[:50000]
    tail = jnp.maximum(0.5 * (b_wr + b_rp), 0.0) @ W_lin + b_lin
    o_p = jnp.concatenate(
        [o_p_head, jnp.broadcast_to(tail, (N_PAPER - 50000, H))])
    return (o_a, o_p, o_u)
